# baseline (speedup 1.0000x reference)
import jax
import jax.numpy as jnp
from jax import lax
from jax.experimental import pallas as pl
from jax.experimental.pallas import tpu as pltpu
from jax.experimental.pallas import tpu_sc as plsc

N = 10000
E = 320000
D_IN = 128
D_EDGE = 16
H = 200
B = 64
T = 12

NC, NS, L = 2, 16, 16
NW = NC * NS
NBLK = 5
ROWS = 16
KB = 128
CHUNK = NBLK * ROWS * KB
EP = NW * CHUNK
NPAD = 10240
DUMMY = 10200
HP = 256
HC = 128
HE = 208
RB = 1024
NRB = NPAD // RB

_MESH = plsc.VectorSubcoreMesh(core_axis_name="c", subcore_axis_name="s",
                               num_cores=NC, num_subcores=NS)
_SC_PARAMS = pltpu.CompilerParams(needs_layout_passes=False)


def _leaky(v):
    return jnp.maximum(v, 0.01 * v)


def _safe_exp(v):
    return jnp.exp(jnp.clip(v, -60.0, 60.0))


def _elu(v):
    return jnp.where(v > 0, v, jnp.exp(v) - 1.0)


def _erf(x):
    ax = jnp.abs(x)
    t = 1.0 / (1.0 + 0.3275911 * ax)
    poly = t * (0.254829592 + t * (-0.284496736 + t * (1.421413741
           + t * (-1.453152027 + t * 1.061405429))))
    y = 1.0 - poly * jnp.exp(-ax * ax)
    return jnp.sign(x) * y


def _gelu(x):
    return 0.5 * x * (1.0 + _erf(x * 0.7071067811865476))


def _gru(xin, h, wih, whh, bih, bhh):
    gi = xin @ wih.T + bih
    gh = h @ whh.T + bhh
    r = jax.nn.sigmoid(gi[:, :H] + gh[:, :H])
    z = jax.nn.sigmoid(gi[:, H:2 * H] + gh[:, H:2 * H])
    n = jnp.tanh(gi[:, 2 * H:] + r * gh[:, 2 * H:])
    return (1.0 - z) * n + z * h


def _sc_alpha_gate_body(asrc_hbm, be_hbm, rdst_hbm, attl_hbm, src_hbm, dst_hbm,
                        ex_hbm, den_hbm,
                        rdst_v, attl_v, src_v, dst_v, ex_v, ag_a, ag_b, bbuf,
                        zv, den_sp, sem_a, sem_b, sem_bb):
    c = lax.axis_index("c")
    s = lax.axis_index("s")
    w = c * NS + s
    pltpu.sync_copy(rdst_hbm, rdst_v)
    pltpu.sync_copy(attl_hbm, attl_v)
    zero16 = jnp.zeros((L,), jnp.float32)
    for i in range(640 // L):
        zv[pl.ds(L * i, L)] = zero16
    pltpu.sync_copy(zv, den_sp.at[pl.ds(640 * s, 640)])
    plsc.subcore_barrier()

    def block(b, carry):
        pltpu.sync_copy(src_hbm.at[w, b], src_v)
        pltpu.sync_copy(dst_hbm.at[w, b], dst_v)

        def process(j, ag, sem):
            bb = pltpu.async_copy(be_hbm.at[w, b, j], bbuf, sem_bb)
            pltpu.make_async_copy(asrc_hbm.at[src_v.at[j]], ag, sem).wait()
            bb.wait()

            def group(g, carry3):
                iota = lax.iota(jnp.int32, L)
                sums = jnp.zeros((L,), jnp.float32)
                for es in range(L):
                    acc = jnp.zeros((L,), jnp.float32)
                    for q in range(HE // L):
                        u = (ag[L * g + es, pl.ds(L * q, L)]
                             + bbuf[L * g + es, pl.ds(L * q, L)])
                        acc = acc + _leaky(u) * attl_v[pl.ds(L * q, L)]
                    sums = jnp.where(iota == es, jnp.sum(acc), sums)
                dv = dst_v[j, pl.ds(L * g, L)]
                a = sums + plsc.load_gather(rdst_v, [dv])
                ex_v[j, pl.ds(L * g, L)] = _safe_exp(_leaky(a))
                return carry3

            lax.fori_loop(0, KB // L, group, 0)

        pltpu.async_copy(asrc_hbm.at[src_v.at[0]], ag_a, sem_a)

        def row2(j2, carry2):
            ja = 2 * j2
            pltpu.async_copy(asrc_hbm.at[src_v.at[ja + 1]], ag_b, sem_b)
            process(ja, ag_a, sem_a)

            @pl.when(j2 < ROWS // 2 - 1)
            def _():
                pltpu.async_copy(asrc_hbm.at[src_v.at[ja + 2]], ag_a, sem_a)

            process(ja + 1, ag_b, sem_b)
            return carry2

        lax.fori_loop(0, ROWS // 2, row2, 0)
        pltpu.sync_copy(ex_v, ex_hbm.at[w, b])

        def rowadd(j, carry2):
            pltpu.sync_copy(ex_v.at[j], den_sp.at[dst_v.at[j]], add=True)
            return carry2

        lax.fori_loop(0, ROWS, rowadd, 0)
        return carry

    lax.fori_loop(0, NBLK, block, 0)
    plsc.subcore_barrier()

    @pl.when(s == 0)
    def _():
        pltpu.sync_copy(den_sp, den_hbm.at[c])


def _sc_alpha_gate(asrcP, be5, rdst, attl, srcp, dstp):
    return pl.kernel(
        _sc_alpha_gate_body,
        out_type=(jax.ShapeDtypeStruct((NW, NBLK, ROWS, KB), jnp.float32),
                  jax.ShapeDtypeStruct((NC, NPAD), jnp.float32)),
        mesh=_MESH,
        compiler_params=_SC_PARAMS,
        scratch_types=[
            pltpu.VMEM((NPAD,), jnp.float32),
            pltpu.VMEM((HE,), jnp.float32),
            pltpu.VMEM((ROWS, KB), jnp.int32),
            pltpu.VMEM((ROWS, KB), jnp.int32),
            pltpu.VMEM((ROWS, KB), jnp.float32),
            pltpu.VMEM((KB, HP), jnp.float32),
            pltpu.VMEM((KB, HP), jnp.float32),
            pltpu.VMEM((KB, HE), jnp.float32),
            pltpu.VMEM((640,), jnp.float32),
            pltpu.VMEM_SHARED((NPAD,), jnp.float32),
            pltpu.SemaphoreType.DMA,
            pltpu.SemaphoreType.DMA,
            pltpu.SemaphoreType.DMA,
        ],
    )(asrcP, be5, rdst, attl, srcp, dstp)


def _sc_alpha_scalar_body(s1_hbm, s2_hbm, src_hbm, dst_hbm,
                          ex_hbm, den_hbm,
                          s1_v, s2_v, src_v, dst_v, ex_v, zv, den_sp):
    c = lax.axis_index("c")
    s = lax.axis_index("s")
    w = c * NS + s
    pltpu.sync_copy(s1_hbm, s1_v)
    pltpu.sync_copy(s2_hbm, s2_v)
    zero16 = jnp.zeros((L,), jnp.float32)
    for i in range(640 // L):
        zv[pl.ds(L * i, L)] = zero16
    pltpu.sync_copy(zv, den_sp.at[pl.ds(640 * s, 640)])
    plsc.subcore_barrier()

    def block(b, carry):
        pltpu.sync_copy(src_hbm.at[w, b], src_v)
        pltpu.sync_copy(dst_hbm.at[w, b], dst_v)

        def row(j, carry2):
            for k in range(KB // L):
                sv = src_v[j, pl.ds(L * k, L)]
                dv = dst_v[j, pl.ds(L * k, L)]
                a = plsc.load_gather(s1_v, [sv]) + plsc.load_gather(s2_v, [dv])
                ex_v[j, pl.ds(L * k, L)] = _safe_exp(_leaky(a))
            return carry2

        lax.fori_loop(0, ROWS, row, 0)
        pltpu.sync_copy(ex_v, ex_hbm.at[w, b])

        def rowadd(j, carry2):
            pltpu.sync_copy(ex_v.at[j], den_sp.at[dst_v.at[j]], add=True)
            return carry2

        lax.fori_loop(0, ROWS, rowadd, 0)
        return carry

    lax.fori_loop(0, NBLK, block, 0)
    plsc.subcore_barrier()

    @pl.when(s == 0)
    def _():
        pltpu.sync_copy(den_sp, den_hbm.at[c])


def _sc_alpha_scalar(s1, s2, srcp, dstp):
    return pl.kernel(
        _sc_alpha_scalar_body,
        out_type=(jax.ShapeDtypeStruct((NW, NBLK, ROWS, KB), jnp.float32),
                  jax.ShapeDtypeStruct((NC, NPAD), jnp.float32)),
        mesh=_MESH,
        compiler_params=_SC_PARAMS,
        scratch_types=[
            pltpu.VMEM((NPAD,), jnp.float32),
            pltpu.VMEM((NPAD,), jnp.float32),
            pltpu.VMEM((ROWS, KB), jnp.int32),
            pltpu.VMEM((ROWS, KB), jnp.int32),
            pltpu.VMEM((ROWS, KB), jnp.float32),
            pltpu.VMEM((640,), jnp.float32),
            pltpu.VMEM_SHARED((NPAD,), jnp.float32),
        ],
    )(s1, s2, srcp, dstp)


def _sc_aggregate_body(tab_hbm, src_hbm, dst_hbm, ex_hbm, den_hbm, z_hbm,
                       acc_hbm,
                       den_v, src_v, dst_v, ex_v, gbuf_a, gbuf_b, acc_sp,
                       gsem_a, gsem_b, ssem_a, ssem_b):
    c = lax.axis_index("c")
    s = lax.axis_index("s")
    pltpu.sync_copy(den_hbm, den_v)
    pltpu.sync_copy(z_hbm, acc_sp.at[pl.ds(640 * s, 640)])
    plsc.subcore_barrier()

    def chunk(half, carry0):
        ch = 2 * s + half

        def block(b, carry):
            pltpu.sync_copy(src_hbm.at[ch, b], src_v)
            pltpu.sync_copy(dst_hbm.at[ch, b], dst_v)
            pltpu.sync_copy(ex_hbm.at[ch, b], ex_v)

            def scale(j, gbuf, gsem):
                pltpu.make_async_copy(tab_hbm.at[c].at[src_v.at[j]], gbuf,
                                      gsem).wait()

                def group(g, carry3):
                    dv = dst_v[j, pl.ds(L * g, L)]
                    exv = ex_v[j, pl.ds(L * g, L)]
                    wv = exv / plsc.load_gather(den_v, [dv])
                    for e in range(L):
                        r = L * g + e
                        we = wv[e]
                        for q in range(HC // L):
                            gbuf[r, pl.ds(L * q, L)] = (
                                we * gbuf[r, pl.ds(L * q, L)])
                    return carry3

                lax.fori_loop(0, KB // L, group, 0)

            def sscat(j, gbuf, ssem):
                pltpu.make_async_copy(gbuf, acc_sp.at[dst_v.at[j]],
                                      ssem).start(add=True)

            def wscat(j, gbuf, ssem):
                pltpu.make_async_copy(gbuf, acc_sp.at[dst_v.at[j]],
                                      ssem).wait()

            pltpu.async_copy(tab_hbm.at[c].at[src_v.at[0]], gbuf_a, gsem_a)

            def row2(j2, carry2):
                ja = 2 * j2

                @pl.when(j2 > 0)
                def _():
                    wscat(ja - 1, gbuf_b, ssem_b)

                pltpu.async_copy(tab_hbm.at[c].at[src_v.at[ja + 1]],
                                 gbuf_b, gsem_b)
                scale(ja, gbuf_a, gsem_a)
                sscat(ja, gbuf_a, ssem_a)
                scale(ja + 1, gbuf_b, gsem_b)
                wscat(ja, gbuf_a, ssem_a)

                @pl.when(j2 < ROWS // 2 - 1)
                def _():
                    pltpu.async_copy(tab_hbm.at[c].at[src_v.at[ja + 2]],
                                     gbuf_a, gsem_a)

                sscat(ja + 1, gbuf_b, ssem_b)
                return carry2

            lax.fori_loop(0, ROWS // 2, row2, 0)
            wscat(ROWS - 1, gbuf_b, ssem_b)
            return carry

        lax.fori_loop(0, NBLK, block, 0)
        return carry0

    lax.fori_loop(0, 2, chunk, 0)
    plsc.subcore_barrier()
    pltpu.sync_copy(acc_sp.at[pl.ds(640 * s, 640)],
                    acc_hbm.at[c, pl.ds(640 * s, 640)])


def _sc_aggregate(tab, srcp, dstp, ex, den):
    zeros = jnp.zeros((640, HC), jnp.float32)
    return pl.kernel(
        _sc_aggregate_body,
        out_type=jax.ShapeDtypeStruct((NC, NPAD, HC), jnp.float32),
        mesh=_MESH,
        compiler_params=_SC_PARAMS,
        scratch_types=[
            pltpu.VMEM((NPAD,), jnp.float32),
            pltpu.VMEM((ROWS, KB), jnp.int32),
            pltpu.VMEM((ROWS, KB), jnp.int32),
            pltpu.VMEM((ROWS, KB), jnp.float32),
            pltpu.VMEM((KB, HC), jnp.float32),
            pltpu.VMEM((KB, HC), jnp.float32),
            pltpu.VMEM_SHARED((NPAD, HC), jnp.float32),
            pltpu.SemaphoreType.DMA,
            pltpu.SemaphoreType.DMA,
            pltpu.SemaphoreType.DMA,
            pltpu.SemaphoreType.DMA,
        ],
    )(tab, srcp, dstp, ex, den, zeros)


def _tc_node_pre_body(x_ref, pw_ref, pb_ref, lg_ref, lb_ref,
                      l1w_ref, l1b_ref, wx_ref, m2_ref, attr_ref,
                      xh_ref, asrc_ref, m2t_ref, rdst_ref):
    xb = x_ref[...]
    h0 = xb @ pw_ref[...].T + pb_ref[...]
    mu = h0.mean(-1, keepdims=True)
    var = ((h0 - mu) ** 2).mean(-1, keepdims=True)
    h0 = (h0 - mu) / jnp.sqrt(var + 1e-5) * lg_ref[...] + lb_ref[...]
    h0 = _gelu(h0)
    xh = _leaky(h0 @ l1w_ref[...].T + l1b_ref[...])
    xh_ref[...] = xh
    asrc_ref[...] = xh @ wx_ref[...].T
    m2p = xh @ m2_ref[...].T
    m2t_ref[0] = m2p[:, :HC]
    m2t_ref[1] = m2p[:, HC:]
    rdst_ref[...] = xh @ attr_ref[...]


def _tc_node_pre(xP, p):
    wxP = jnp.pad(p["gate_lin1_W"][:, :H], ((0, HP - H), (0, 0)))
    m2P = jnp.pad(p["gate_lin2_W"], ((0, HP - H), (0, 0)))
    full = lambda a: pl.BlockSpec(a, lambda i: tuple(0 for _ in a))
    return pl.pallas_call(
        _tc_node_pre_body,
        grid=(NRB,),
        in_specs=[
            pl.BlockSpec((RB, D_IN), lambda i: (i, 0)),
            full((H, D_IN)), full((H,)), full((H,)), full((H,)),
            full((H, H)), full((H,)), full((HP, H)), full((HP, H)),
            full((H, 1)),
        ],
        out_specs=[
            pl.BlockSpec((RB, H), lambda i: (i, 0)),
            pl.BlockSpec((RB, HP), lambda i: (i, 0)),
            pl.BlockSpec((2, RB, HC), lambda i: (0, i, 0)),
            pl.BlockSpec((RB, 1), lambda i: (i, 0)),
        ],
        out_shape=[
            jax.ShapeDtypeStruct((NPAD, H), jnp.float32),
            jax.ShapeDtypeStruct((NPAD, HP), jnp.float32),
            jax.ShapeDtypeStruct((NC, NPAD, HC), jnp.float32),
            jax.ShapeDtypeStruct((NPAD, 1), jnp.float32),
        ],
    )(xP, p["proj_W"], p["proj_b"], p["ln_g"], p["ln_b"],
      p["lin1_W"], p["lin1_b"], wxP, m2P, p["gate_att_r"][:, None])


def _tc_edge_mlp_body(ea_ref, we_ref, out_ref):
    out_ref[...] = ea_ref[...] @ we_ref[...].T


def _tc_edge_mlp(eaP, p):
    weP = jnp.pad(p["gate_lin1_W"][:, H:], ((0, HE - H), (0, 0)))
    return pl.pallas_call(
        _tc_edge_mlp_body,
        grid=(EP // RB,),
        in_specs=[
            pl.BlockSpec((RB, D_EDGE), lambda i: (i, 0)),
            pl.BlockSpec((HE, D_EDGE), lambda i: (0, 0)),
        ],
        out_specs=pl.BlockSpec((RB, HE), lambda i: (i, 0)),
        out_shape=jax.ShapeDtypeStruct((EP, HE), jnp.float32),
    )(eaP, weP)


def _tc_post_gate_body(acc_ref, xh_ref, gb_ref, wih_ref, whh_ref, bih_ref,
                       bhh_ref, c1_ref, as_ref, ad_ref,
                       xh1_ref, xst_ref, s1_ref, s2_ref):
    acc0 = acc_ref[0]
    acc1 = acc_ref[1]
    agg = jnp.concatenate([acc0, acc1[:, :H - HC]], axis=1)
    h = _elu(agg + gb_ref[...])
    xh = xh_ref[...]
    xh1 = jax.nn.relu(_gru(h, xh, wih_ref[...], whh_ref[...],
                           bih_ref[...], bhh_ref[...]))
    xh1_ref[...] = xh1
    xsp = xh1 @ c1_ref[...].T
    xst_ref[0] = xsp[:, :HC]
    xst_ref[1] = xsp[:, HC:]
    s1_ref[...] = xsp[:, :H] @ as_ref[...]
    s2_ref[...] = xsp[:, :H] @ ad_ref[...]


def _tc_post_gate(accG, xh, p):
    c1P = jnp.pad(p["conv1_W"], ((0, HP - H), (0, 0)))
    g = p["gru0"]
    full = lambda a: pl.BlockSpec(a, lambda i: tuple(0 for _ in a))
    return pl.pallas_call(
        _tc_post_gate_body,
        grid=(NRB,),
        in_specs=[
            pl.BlockSpec((2, RB, HC), lambda i: (0, i, 0)),
            pl.BlockSpec((RB, H), lambda i: (i, 0)),
            full((H,)), full((3 * H, H)), full((3 * H, H)),
            full((3 * H,)), full((3 * H,)),
            full((HP, H)), full((H, 1)), full((H, 1)),
        ],
        out_specs=[
            pl.BlockSpec((RB, H), lambda i: (i, 0)),
            pl.BlockSpec((2, RB, HC), lambda i: (0, i, 0)),
            pl.BlockSpec((RB, 1), lambda i: (i, 0)),
            pl.BlockSpec((RB, 1), lambda i: (i, 0)),
        ],
        out_shape=[
            jax.ShapeDtypeStruct((NPAD, H), jnp.float32),
            jax.ShapeDtypeStruct((NC, NPAD, HC), jnp.float32),
            jax.ShapeDtypeStruct((NPAD, 1), jnp.float32),
            jax.ShapeDtypeStruct((NPAD, 1), jnp.float32),
        ],
    )(accG, xh, p["gate_bias"], g["Wih"], g["Whh"], g["bih"], g["bhh"],
      c1P, p["conv1_att_src"][:, None], p["conv1_att_dst"][:, None])


def _tc_post_conv_body(acc_ref, xh1_ref, cb_ref, wih_ref, whh_ref, bih_ref,
                       bhh_ref, mw_ref, br_ref,
                       xsn_ref, g0_ref):
    i = pl.program_id(0)
    acc0 = acc_ref[0]
    acc1 = acc_ref[1]
    agg = jnp.concatenate([acc0, acc1[:, :H - HC]], axis=1)
    h = jax.nn.relu(agg + cb_ref[...])
    xh1 = xh1_ref[...]
    xh2 = jax.nn.relu(_gru(h, xh1, wih_ref[...], whh_ref[...],
                           bih_ref[...], bhh_ref[...]))
    xsn_ref[...] = xh2 @ mw_ref[...].T
    bb = br_ref[...]
    iot = lax.broadcasted_iota(jnp.int32, (B, RB), 0)
    oh = (bb == iot).astype(jnp.float32)

    @pl.when(i == 0)
    def _():
        g0_ref[...] = jnp.zeros((B, H), jnp.float32)

    g0_ref[...] += oh @ xh2


def _tc_post_conv(accC, xh1, batch_row, p):
    g = p["gru1"]
    full = lambda a: pl.BlockSpec(a, lambda i: tuple(0 for _ in a))
    return pl.pallas_call(
        _tc_post_conv_body,
        grid=(NRB,),
        in_specs=[
            pl.BlockSpec((2, RB, HC), lambda i: (0, i, 0)),
            pl.BlockSpec((RB, H), lambda i: (i, 0)),
            full((H,)), full((3 * H, H)), full((3 * H, H)),
            full((3 * H,)), full((3 * H,)), full((H, H)),
            pl.BlockSpec((1, RB), lambda i: (0, i)),
        ],
        out_specs=[
            pl.BlockSpec((RB, H), lambda i: (i, 0)),
            pl.BlockSpec((B, H), lambda i: (0, 0)),
        ],
        out_shape=[
            jax.ShapeDtypeStruct((NPAD, H), jnp.float32),
            jax.ShapeDtypeStruct((B, H), jnp.float32),
        ],
    )(accC, xh1, p["conv1_bias"], g["Wih"], g["Whh"], g["bih"], g["bhh"],
      p["mol_W"], batch_row)


def _tc_readout_body(xsn_ref, br_ref, bc_ref, g0_ref,
                     mw_ref, mad_ref, mas_ref, mb_ref,
                     wih_ref, whh_ref, bih_ref, bhh_ref,
                     w2_ref, b2_ref, t1w_ref, t1b_ref, t2w_ref, t2b_ref,
                     hw_ref, hb_ref, out_ref):
    xsn = xsn_ref[...]
    asm = xsn @ mas_ref[...]
    br = br_ref[...]
    bc = bc_ref[...]
    iot_r = lax.broadcasted_iota(jnp.int32, (B, NPAD), 0)
    oh = (br == iot_r).astype(jnp.float32)
    iot_c = lax.broadcasted_iota(jnp.int32, (NPAD, B), 1)
    oht = (bc == iot_c).astype(jnp.float32)
    g = jax.nn.relu(g0_ref[...])
    for _ in range(2):
        gs = g @ mw_ref[...].T
        av = gs @ mad_ref[...]
        alpha = _leaky(asm + oht @ av)
        alpha_r = alpha.T
        aw = jnp.where(oh > 0, alpha_r, -1e30)
        amax = aw.max(axis=1, keepdims=True)
        amax = jnp.where(amax > -1e29, amax, 0.0)
        exm = oh * jnp.exp(alpha_r - amax)
        den = exm.sum(axis=1, keepdims=True)
        num = exm @ xsn
        h = _elu(num / (den + 1e-16) + mb_ref[...])
        g = jax.nn.relu(_gru(h, g, wih_ref[...], whh_ref[...],
                             bih_ref[...], bhh_ref[...]))
    out = g @ w2_ref[...].T + b2_ref[...]
    t1 = _gelu(out @ t1w_ref[...].T + t1b_ref[...])
    t2 = _gelu(t1 @ t2w_ref[...].T + t2b_ref[...])
    out_ref[...] = t2 @ hw_ref[...].T + hb_ref[...]


def _tc_readout(xsn, batch_row, batch_col, g0, p):
    g = p["mgru"]
    return pl.pallas_call(
        _tc_readout_body,
        out_shape=jax.ShapeDtypeStruct((B, T), jnp.float32),
    )(xsn, batch_row, batch_col, g0,
      p["mol_W"], p["mol_att_dst"][:, None], p["mol_att_src"][:, None],
      p["mol_bias"],
      g["Wih"], g["Whh"], g["bih"], g["bhh"],
      p["lin2_W"], p["lin2_b"], p["trunk1_W"], p["trunk1_b"],
      p["trunk2_W"], p["trunk2_b"], p["head_W"], p["head_b"])


def _pad_edges(idx, fill):
    a = idx.reshape(NW, E // NW)
    a = jnp.pad(a, ((0, 0), (0, CHUNK - E // NW)), constant_values=fill)
    return a.reshape(NW, NBLK, ROWS, KB)


def kernel(x, edge_index, edge_attr, batch, params):
    p = params
    srcp = _pad_edges(edge_index[0], 0)
    dstp = _pad_edges(edge_index[1], DUMMY)
    xP = jnp.pad(x, ((0, NPAD - N), (0, 0)))
    eaP = jnp.pad(edge_attr.reshape(NW, E // NW, D_EDGE),
                  ((0, 0), (0, CHUNK - E // NW), (0, 0))).reshape(EP, D_EDGE)
    batchP = jnp.pad(batch, (0, NPAD - N), constant_values=127)
    batch_row = batchP.reshape(1, NPAD)
    batch_col = batchP.reshape(NPAD, 1)
    attlP = jnp.pad(p["gate_att_l"], (0, HE - H))

    xh, asrcP, m2T, rdst = _tc_node_pre(xP, p)
    bE = _tc_edge_mlp(eaP, p).reshape(NW, NBLK, ROWS, KB, HE)
    exG, denG = _sc_alpha_gate(asrcP, bE, rdst.reshape(NPAD), attlP,
                               srcp, dstp)
    accG = _sc_aggregate(m2T, srcp, dstp, exG, denG[0] + denG[1])
    xh1, xsT, s1, s2 = _tc_post_gate(accG, xh, p)
    exC, denC = _sc_alpha_scalar(s1.reshape(NPAD), s2.reshape(NPAD),
                                 srcp, dstp)
    accC = _sc_aggregate(xsT, srcp, dstp, exC, denC[0] + denC[1])
    xsn, g0 = _tc_post_conv(accC, xh1, batch_row, p)
    return _tc_readout(xsn, batch_row, batch_col, g0, p)


# speedup vs baseline: 1.0118x; 1.0118x over previous
import jax
import jax.numpy as jnp
from jax import lax
from jax.experimental import pallas as pl
from jax.experimental.pallas import tpu as pltpu
from jax.experimental.pallas import tpu_sc as plsc

N = 10000
E = 320000
D_IN = 128
D_EDGE = 16
H = 200
B = 64
T = 12

NC, NS, L = 2, 16, 16
NW = NC * NS
NBLK = 5
ROWS = 16
KB = 128
CHUNK = NBLK * ROWS * KB
EP = NW * CHUNK
NPAD = 10240
DUMMY = 10200
HP = 256
HC = 128
HE = 208
RB = 1024
NRB = NPAD // RB

_MESH = plsc.VectorSubcoreMesh(core_axis_name="c", subcore_axis_name="s",
                               num_cores=NC, num_subcores=NS)
_SC_PARAMS = pltpu.CompilerParams(needs_layout_passes=False)


def _leaky(v):
    return jnp.maximum(v, 0.01 * v)


def _safe_exp(v):
    return jnp.exp(jnp.clip(v, -60.0, 60.0))


def _elu(v):
    return jnp.where(v > 0, v, jnp.exp(v) - 1.0)


def _erf(x):
    ax = jnp.abs(x)
    t = 1.0 / (1.0 + 0.3275911 * ax)
    poly = t * (0.254829592 + t * (-0.284496736 + t * (1.421413741
           + t * (-1.453152027 + t * 1.061405429))))
    y = 1.0 - poly * jnp.exp(-ax * ax)
    return jnp.sign(x) * y


def _gelu(x):
    return 0.5 * x * (1.0 + _erf(x * 0.7071067811865476))


def _gru(xin, h, wih, whh, bih, bhh):
    gi = xin @ wih.T + bih
    gh = h @ whh.T + bhh
    r = jax.nn.sigmoid(gi[:, :H] + gh[:, :H])
    z = jax.nn.sigmoid(gi[:, H:2 * H] + gh[:, H:2 * H])
    n = jnp.tanh(gi[:, 2 * H:] + r * gh[:, 2 * H:])
    return (1.0 - z) * n + z * h


def _sc_alpha_gate_body(asrc_hbm, be_hbm, rdst_hbm, attl_hbm, src_hbm, dst_hbm,
                        ex_hbm, den_hbm,
                        rdst_v, attl_v, src_v, dst_v, ex_v, ag_a, ag_b, bbuf,
                        zv, den_sp, sem_a, sem_b, sem_bb):
    c = lax.axis_index("c")
    s = lax.axis_index("s")
    w = c * NS + s
    pltpu.sync_copy(rdst_hbm, rdst_v)
    pltpu.sync_copy(attl_hbm, attl_v)
    zero16 = jnp.zeros((L,), jnp.float32)
    for i in range(640 // L):
        zv[pl.ds(L * i, L)] = zero16
    pltpu.sync_copy(zv, den_sp.at[pl.ds(640 * s, 640)])
    plsc.subcore_barrier()

    def block(b, carry):
        pltpu.sync_copy(src_hbm.at[w, b], src_v)
        pltpu.sync_copy(dst_hbm.at[w, b], dst_v)

        def process(j, ag, sem):
            bb = pltpu.async_copy(be_hbm.at[w, b, j], bbuf, sem_bb)
            pltpu.make_async_copy(asrc_hbm.at[src_v.at[j]], ag, sem).wait()
            bb.wait()

            def group(g, carry3):
                iota = lax.iota(jnp.int32, L)
                sums = jnp.zeros((L,), jnp.float32)
                for es in range(L):
                    accs = [jnp.zeros((L,), jnp.float32) for _ in range(4)]
                    for q in range(HE // L):
                        u = (ag[L * g + es, pl.ds(L * q, L)]
                             + bbuf[L * g + es, pl.ds(L * q, L)])
                        accs[q % 4] = (accs[q % 4]
                                       + _leaky(u) * attl_v[pl.ds(L * q, L)])
                    acc = (accs[0] + accs[1]) + (accs[2] + accs[3])
                    sums = jnp.where(iota == es, jnp.sum(acc), sums)
                dv = dst_v[j, pl.ds(L * g, L)]
                a = sums + plsc.load_gather(rdst_v, [dv])
                ex_v[j, pl.ds(L * g, L)] = _safe_exp(_leaky(a))
                return carry3

            lax.fori_loop(0, KB // L, group, 0)

        pltpu.async_copy(asrc_hbm.at[src_v.at[0]], ag_a, sem_a)

        def row2(j2, carry2):
            ja = 2 * j2
            pltpu.async_copy(asrc_hbm.at[src_v.at[ja + 1]], ag_b, sem_b)
            process(ja, ag_a, sem_a)

            @pl.when(j2 < ROWS // 2 - 1)
            def _():
                pltpu.async_copy(asrc_hbm.at[src_v.at[ja + 2]], ag_a, sem_a)

            process(ja + 1, ag_b, sem_b)
            return carry2

        lax.fori_loop(0, ROWS // 2, row2, 0)
        pltpu.sync_copy(ex_v, ex_hbm.at[w, b])

        def rowadd(j, carry2):
            pltpu.sync_copy(ex_v.at[j], den_sp.at[dst_v.at[j]], add=True)
            return carry2

        lax.fori_loop(0, ROWS, rowadd, 0)
        return carry

    lax.fori_loop(0, NBLK, block, 0)
    plsc.subcore_barrier()

    @pl.when(s == 0)
    def _():
        pltpu.sync_copy(den_sp, den_hbm.at[c])


def _sc_alpha_gate(asrcP, be5, rdst, attl, srcp, dstp):
    return pl.kernel(
        _sc_alpha_gate_body,
        out_type=(jax.ShapeDtypeStruct((NW, NBLK, ROWS, KB), jnp.float32),
                  jax.ShapeDtypeStruct((NC, NPAD), jnp.float32)),
        mesh=_MESH,
        compiler_params=_SC_PARAMS,
        scratch_types=[
            pltpu.VMEM((NPAD,), jnp.float32),
            pltpu.VMEM((HE,), jnp.float32),
            pltpu.VMEM((ROWS, KB), jnp.int32),
            pltpu.VMEM((ROWS, KB), jnp.int32),
            pltpu.VMEM((ROWS, KB), jnp.float32),
            pltpu.VMEM((KB, HP), jnp.float32),
            pltpu.VMEM((KB, HP), jnp.float32),
            pltpu.VMEM((KB, HE), jnp.float32),
            pltpu.VMEM((640,), jnp.float32),
            pltpu.VMEM_SHARED((NPAD,), jnp.float32),
            pltpu.SemaphoreType.DMA,
            pltpu.SemaphoreType.DMA,
            pltpu.SemaphoreType.DMA,
        ],
    )(asrcP, be5, rdst, attl, srcp, dstp)


def _sc_alpha_scalar_body(s1_hbm, s2_hbm, src_hbm, dst_hbm,
                          ex_hbm, den_hbm,
                          s1_v, s2_v, src_v, dst_v, ex_v, zv, den_sp):
    c = lax.axis_index("c")
    s = lax.axis_index("s")
    w = c * NS + s
    pltpu.sync_copy(s1_hbm, s1_v)
    pltpu.sync_copy(s2_hbm, s2_v)
    zero16 = jnp.zeros((L,), jnp.float32)
    for i in range(640 // L):
        zv[pl.ds(L * i, L)] = zero16
    pltpu.sync_copy(zv, den_sp.at[pl.ds(640 * s, 640)])
    plsc.subcore_barrier()

    def block(b, carry):
        pltpu.sync_copy(src_hbm.at[w, b], src_v)
        pltpu.sync_copy(dst_hbm.at[w, b], dst_v)

        def row(j, carry2):
            for k in range(KB // L):
                sv = src_v[j, pl.ds(L * k, L)]
                dv = dst_v[j, pl.ds(L * k, L)]
                a = plsc.load_gather(s1_v, [sv]) + plsc.load_gather(s2_v, [dv])
                ex_v[j, pl.ds(L * k, L)] = _safe_exp(_leaky(a))
            return carry2

        lax.fori_loop(0, ROWS, row, 0)
        pltpu.sync_copy(ex_v, ex_hbm.at[w, b])

        def rowadd(j, carry2):
            pltpu.sync_copy(ex_v.at[j], den_sp.at[dst_v.at[j]], add=True)
            return carry2

        lax.fori_loop(0, ROWS, rowadd, 0)
        return carry

    lax.fori_loop(0, NBLK, block, 0)
    plsc.subcore_barrier()

    @pl.when(s == 0)
    def _():
        pltpu.sync_copy(den_sp, den_hbm.at[c])


def _sc_alpha_scalar(s1, s2, srcp, dstp):
    return pl.kernel(
        _sc_alpha_scalar_body,
        out_type=(jax.ShapeDtypeStruct((NW, NBLK, ROWS, KB), jnp.float32),
                  jax.ShapeDtypeStruct((NC, NPAD), jnp.float32)),
        mesh=_MESH,
        compiler_params=_SC_PARAMS,
        scratch_types=[
            pltpu.VMEM((NPAD,), jnp.float32),
            pltpu.VMEM((NPAD,), jnp.float32),
            pltpu.VMEM((ROWS, KB), jnp.int32),
            pltpu.VMEM((ROWS, KB), jnp.int32),
            pltpu.VMEM((ROWS, KB), jnp.float32),
            pltpu.VMEM((640,), jnp.float32),
            pltpu.VMEM_SHARED((NPAD,), jnp.float32),
        ],
    )(s1, s2, srcp, dstp)


def _sc_aggregate_body(tab_hbm, src_hbm, dst_hbm, ex_hbm, den_hbm, z_hbm,
                       acc_hbm,
                       den_v, src_v, dst_v, ex_v, gbuf_a, gbuf_b, acc_sp,
                       gsem_a, gsem_b, ssem_a, ssem_b):
    c = lax.axis_index("c")
    s = lax.axis_index("s")
    pltpu.sync_copy(den_hbm, den_v)
    pltpu.sync_copy(z_hbm, acc_sp.at[pl.ds(640 * s, 640)])
    plsc.subcore_barrier()

    def chunk(half, carry0):
        ch = 2 * s + half

        def block(b, carry):
            pltpu.sync_copy(src_hbm.at[ch, b], src_v)
            pltpu.sync_copy(dst_hbm.at[ch, b], dst_v)
            pltpu.sync_copy(ex_hbm.at[ch, b], ex_v)

            def scale(j, gbuf, gsem):
                pltpu.make_async_copy(tab_hbm.at[c].at[src_v.at[j]], gbuf,
                                      gsem).wait()

                def group(g, carry3):
                    dv = dst_v[j, pl.ds(L * g, L)]
                    exv = ex_v[j, pl.ds(L * g, L)]
                    wv = exv / plsc.load_gather(den_v, [dv])
                    for e in range(L):
                        r = L * g + e
                        we = wv[e]
                        for q in range(HC // L):
                            gbuf[r, pl.ds(L * q, L)] = (
                                we * gbuf[r, pl.ds(L * q, L)])
                    return carry3

                lax.fori_loop(0, KB // L, group, 0)

            def sscat(j, gbuf, ssem):
                pltpu.make_async_copy(gbuf, acc_sp.at[dst_v.at[j]],
                                      ssem).start(add=True)

            def wscat(j, gbuf, ssem):
                pltpu.make_async_copy(gbuf, acc_sp.at[dst_v.at[j]],
                                      ssem).wait()

            pltpu.async_copy(tab_hbm.at[c].at[src_v.at[0]], gbuf_a, gsem_a)

            def row2(j2, carry2):
                ja = 2 * j2

                @pl.when(j2 > 0)
                def _():
                    wscat(ja - 1, gbuf_b, ssem_b)

                pltpu.async_copy(tab_hbm.at[c].at[src_v.at[ja + 1]],
                                 gbuf_b, gsem_b)
                scale(ja, gbuf_a, gsem_a)
                sscat(ja, gbuf_a, ssem_a)
                scale(ja + 1, gbuf_b, gsem_b)
                wscat(ja, gbuf_a, ssem_a)

                @pl.when(j2 < ROWS // 2 - 1)
                def _():
                    pltpu.async_copy(tab_hbm.at[c].at[src_v.at[ja + 2]],
                                     gbuf_a, gsem_a)

                sscat(ja + 1, gbuf_b, ssem_b)
                return carry2

            lax.fori_loop(0, ROWS // 2, row2, 0)
            wscat(ROWS - 1, gbuf_b, ssem_b)
            return carry

        lax.fori_loop(0, NBLK, block, 0)
        return carry0

    lax.fori_loop(0, 2, chunk, 0)
    plsc.subcore_barrier()
    pltpu.sync_copy(acc_sp.at[pl.ds(640 * s, 640)],
                    acc_hbm.at[c, pl.ds(640 * s, 640)])


def _sc_aggregate(tab, srcp, dstp, ex, den):
    zeros = jnp.zeros((640, HC), jnp.float32)
    return pl.kernel(
        _sc_aggregate_body,
        out_type=jax.ShapeDtypeStruct((NC, NPAD, HC), jnp.float32),
        mesh=_MESH,
        compiler_params=_SC_PARAMS,
        scratch_types=[
            pltpu.VMEM((NPAD,), jnp.float32),
            pltpu.VMEM((ROWS, KB), jnp.int32),
            pltpu.VMEM((ROWS, KB), jnp.int32),
            pltpu.VMEM((ROWS, KB), jnp.float32),
            pltpu.VMEM((KB, HC), jnp.float32),
            pltpu.VMEM((KB, HC), jnp.float32),
            pltpu.VMEM_SHARED((NPAD, HC), jnp.float32),
            pltpu.SemaphoreType.DMA,
            pltpu.SemaphoreType.DMA,
            pltpu.SemaphoreType.DMA,
            pltpu.SemaphoreType.DMA,
        ],
    )(tab, srcp, dstp, ex, den, zeros)


def _tc_node_pre_body(x_ref, pw_ref, pb_ref, lg_ref, lb_ref,
                      l1w_ref, l1b_ref, wx_ref, m2_ref, attr_ref,
                      xh_ref, asrc_ref, m2t_ref, rdst_ref):
    xb = x_ref[...]
    h0 = xb @ pw_ref[...].T + pb_ref[...]
    mu = h0.mean(-1, keepdims=True)
    var = ((h0 - mu) ** 2).mean(-1, keepdims=True)
    h0 = (h0 - mu) / jnp.sqrt(var + 1e-5) * lg_ref[...] + lb_ref[...]
    h0 = _gelu(h0)
    xh = _leaky(h0 @ l1w_ref[...].T + l1b_ref[...])
    xh_ref[...] = xh
    asrc_ref[...] = xh @ wx_ref[...].T
    m2p = xh @ m2_ref[...].T
    m2t_ref[0] = m2p[:, :HC]
    m2t_ref[1] = m2p[:, HC:]
    rdst_ref[...] = xh @ attr_ref[...]


def _tc_node_pre(xP, p):
    wxP = jnp.pad(p["gate_lin1_W"][:, :H], ((0, HP - H), (0, 0)))
    m2P = jnp.pad(p["gate_lin2_W"], ((0, HP - H), (0, 0)))
    full = lambda a: pl.BlockSpec(a, lambda i: tuple(0 for _ in a))
    return pl.pallas_call(
        _tc_node_pre_body,
        grid=(NRB,),
        in_specs=[
            pl.BlockSpec((RB, D_IN), lambda i: (i, 0)),
            full((H, D_IN)), full((H,)), full((H,)), full((H,)),
            full((H, H)), full((H,)), full((HP, H)), full((HP, H)),
            full((H, 1)),
        ],
        out_specs=[
            pl.BlockSpec((RB, H), lambda i: (i, 0)),
            pl.BlockSpec((RB, HP), lambda i: (i, 0)),
            pl.BlockSpec((2, RB, HC), lambda i: (0, i, 0)),
            pl.BlockSpec((RB, 1), lambda i: (i, 0)),
        ],
        out_shape=[
            jax.ShapeDtypeStruct((NPAD, H), jnp.float32),
            jax.ShapeDtypeStruct((NPAD, HP), jnp.float32),
            jax.ShapeDtypeStruct((NC, NPAD, HC), jnp.float32),
            jax.ShapeDtypeStruct((NPAD, 1), jnp.float32),
        ],
    )(xP, p["proj_W"], p["proj_b"], p["ln_g"], p["ln_b"],
      p["lin1_W"], p["lin1_b"], wxP, m2P, p["gate_att_r"][:, None])


def _tc_edge_mlp_body(ea_ref, we_ref, out_ref):
    out_ref[...] = ea_ref[...] @ we_ref[...].T


def _tc_edge_mlp(eaP, p):
    weP = jnp.pad(p["gate_lin1_W"][:, H:], ((0, HE - H), (0, 0)))
    return pl.pallas_call(
        _tc_edge_mlp_body,
        grid=(EP // RB,),
        in_specs=[
            pl.BlockSpec((RB, D_EDGE), lambda i: (i, 0)),
            pl.BlockSpec((HE, D_EDGE), lambda i: (0, 0)),
        ],
        out_specs=pl.BlockSpec((RB, HE), lambda i: (i, 0)),
        out_shape=jax.ShapeDtypeStruct((EP, HE), jnp.float32),
    )(eaP, weP)


def _tc_post_gate_body(acc_ref, xh_ref, gb_ref, wih_ref, whh_ref, bih_ref,
                       bhh_ref, c1_ref, as_ref, ad_ref,
                       xh1_ref, xst_ref, s1_ref, s2_ref):
    acc0 = acc_ref[0]
    acc1 = acc_ref[1]
    agg = jnp.concatenate([acc0, acc1[:, :H - HC]], axis=1)
    h = _elu(agg + gb_ref[...])
    xh = xh_ref[...]
    xh1 = jax.nn.relu(_gru(h, xh, wih_ref[...], whh_ref[...],
                           bih_ref[...], bhh_ref[...]))
    xh1_ref[...] = xh1
    xsp = xh1 @ c1_ref[...].T
    xst_ref[0] = xsp[:, :HC]
    xst_ref[1] = xsp[:, HC:]
    s1_ref[...] = xsp[:, :H] @ as_ref[...]
    s2_ref[...] = xsp[:, :H] @ ad_ref[...]


def _tc_post_gate(accG, xh, p):
    c1P = jnp.pad(p["conv1_W"], ((0, HP - H), (0, 0)))
    g = p["gru0"]
    full = lambda a: pl.BlockSpec(a, lambda i: tuple(0 for _ in a))
    return pl.pallas_call(
        _tc_post_gate_body,
        grid=(NRB,),
        in_specs=[
            pl.BlockSpec((2, RB, HC), lambda i: (0, i, 0)),
            pl.BlockSpec((RB, H), lambda i: (i, 0)),
            full((H,)), full((3 * H, H)), full((3 * H, H)),
            full((3 * H,)), full((3 * H,)),
            full((HP, H)), full((H, 1)), full((H, 1)),
        ],
        out_specs=[
            pl.BlockSpec((RB, H), lambda i: (i, 0)),
            pl.BlockSpec((2, RB, HC), lambda i: (0, i, 0)),
            pl.BlockSpec((RB, 1), lambda i: (i, 0)),
            pl.BlockSpec((RB, 1), lambda i: (i, 0)),
        ],
        out_shape=[
            jax.ShapeDtypeStruct((NPAD, H), jnp.float32),
            jax.ShapeDtypeStruct((NC, NPAD, HC), jnp.float32),
            jax.ShapeDtypeStruct((NPAD, 1), jnp.float32),
            jax.ShapeDtypeStruct((NPAD, 1), jnp.float32),
        ],
    )(accG, xh, p["gate_bias"], g["Wih"], g["Whh"], g["bih"], g["bhh"],
      c1P, p["conv1_att_src"][:, None], p["conv1_att_dst"][:, None])


def _tc_post_conv_body(acc_ref, xh1_ref, cb_ref, wih_ref, whh_ref, bih_ref,
                       bhh_ref, mw_ref, br_ref,
                       xsn_ref, g0_ref):
    i = pl.program_id(0)
    acc0 = acc_ref[0]
    acc1 = acc_ref[1]
    agg = jnp.concatenate([acc0, acc1[:, :H - HC]], axis=1)
    h = jax.nn.relu(agg + cb_ref[...])
    xh1 = xh1_ref[...]
    xh2 = jax.nn.relu(_gru(h, xh1, wih_ref[...], whh_ref[...],
                           bih_ref[...], bhh_ref[...]))
    xsn_ref[...] = xh2 @ mw_ref[...].T
    bb = br_ref[...]
    iot = lax.broadcasted_iota(jnp.int32, (B, RB), 0)
    oh = (bb == iot).astype(jnp.float32)

    @pl.when(i == 0)
    def _():
        g0_ref[...] = jnp.zeros((B, H), jnp.float32)

    g0_ref[...] += oh @ xh2


def _tc_post_conv(accC, xh1, batch_row, p):
    g = p["gru1"]
    full = lambda a: pl.BlockSpec(a, lambda i: tuple(0 for _ in a))
    return pl.pallas_call(
        _tc_post_conv_body,
        grid=(NRB,),
        in_specs=[
            pl.BlockSpec((2, RB, HC), lambda i: (0, i, 0)),
            pl.BlockSpec((RB, H), lambda i: (i, 0)),
            full((H,)), full((3 * H, H)), full((3 * H, H)),
            full((3 * H,)), full((3 * H,)), full((H, H)),
            pl.BlockSpec((1, RB), lambda i: (0, i)),
        ],
        out_specs=[
            pl.BlockSpec((RB, H), lambda i: (i, 0)),
            pl.BlockSpec((B, H), lambda i: (0, 0)),
        ],
        out_shape=[
            jax.ShapeDtypeStruct((NPAD, H), jnp.float32),
            jax.ShapeDtypeStruct((B, H), jnp.float32),
        ],
    )(accC, xh1, p["conv1_bias"], g["Wih"], g["Whh"], g["bih"], g["bhh"],
      p["mol_W"], batch_row)


def _tc_readout_body(xsn_ref, br_ref, bc_ref, g0_ref,
                     mw_ref, mad_ref, mas_ref, mb_ref,
                     wih_ref, whh_ref, bih_ref, bhh_ref,
                     w2_ref, b2_ref, t1w_ref, t1b_ref, t2w_ref, t2b_ref,
                     hw_ref, hb_ref, out_ref):
    xsn = xsn_ref[...]
    asm = xsn @ mas_ref[...]
    br = br_ref[...]
    bc = bc_ref[...]
    iot_r = lax.broadcasted_iota(jnp.int32, (B, NPAD), 0)
    oh = (br == iot_r).astype(jnp.float32)
    iot_c = lax.broadcasted_iota(jnp.int32, (NPAD, B), 1)
    oht = (bc == iot_c).astype(jnp.float32)
    g = jax.nn.relu(g0_ref[...])
    for _ in range(2):
        gs = g @ mw_ref[...].T
        av = gs @ mad_ref[...]
        alpha = _leaky(asm + oht @ av)
        alpha_r = alpha.T
        aw = jnp.where(oh > 0, alpha_r, -1e30)
        amax = aw.max(axis=1, keepdims=True)
        amax = jnp.where(amax > -1e29, amax, 0.0)
        exm = oh * jnp.exp(alpha_r - amax)
        den = exm.sum(axis=1, keepdims=True)
        num = exm @ xsn
        h = _elu(num / (den + 1e-16) + mb_ref[...])
        g = jax.nn.relu(_gru(h, g, wih_ref[...], whh_ref[...],
                             bih_ref[...], bhh_ref[...]))
    out = g @ w2_ref[...].T + b2_ref[...]
    t1 = _gelu(out @ t1w_ref[...].T + t1b_ref[...])
    t2 = _gelu(t1 @ t2w_ref[...].T + t2b_ref[...])
    out_ref[...] = t2 @ hw_ref[...].T + hb_ref[...]


def _tc_readout(xsn, batch_row, batch_col, g0, p):
    g = p["mgru"]
    return pl.pallas_call(
        _tc_readout_body,
        out_shape=jax.ShapeDtypeStruct((B, T), jnp.float32),
    )(xsn, batch_row, batch_col, g0,
      p["mol_W"], p["mol_att_dst"][:, None], p["mol_att_src"][:, None],
      p["mol_bias"],
      g["Wih"], g["Whh"], g["bih"], g["bhh"],
      p["lin2_W"], p["lin2_b"], p["trunk1_W"], p["trunk1_b"],
      p["trunk2_W"], p["trunk2_b"], p["head_W"], p["head_b"])


def _pad_edges(idx, fill):
    a = idx.reshape(NW, E // NW)
    a = jnp.pad(a, ((0, 0), (0, CHUNK - E // NW)), constant_values=fill)
    return a.reshape(NW, NBLK, ROWS, KB)


def kernel(x, edge_index, edge_attr, batch, params):
    p = params
    srcp = _pad_edges(edge_index[0], 0)
    dstp = _pad_edges(edge_index[1], DUMMY)
    xP = jnp.pad(x, ((0, NPAD - N), (0, 0)))
    eaP = jnp.pad(edge_attr.reshape(NW, E // NW, D_EDGE),
                  ((0, 0), (0, CHUNK - E // NW), (0, 0))).reshape(EP, D_EDGE)
    batchP = jnp.pad(batch, (0, NPAD - N), constant_values=127)
    batch_row = batchP.reshape(1, NPAD)
    batch_col = batchP.reshape(NPAD, 1)
    attlP = jnp.pad(p["gate_att_l"], (0, HE - H))

    xh, asrcP, m2T, rdst = _tc_node_pre(xP, p)
    bE = _tc_edge_mlp(eaP, p).reshape(NW, NBLK, ROWS, KB, HE)
    exG, denG = _sc_alpha_gate(asrcP, bE, rdst.reshape(NPAD), attlP,
                               srcp, dstp)
    accG = _sc_aggregate(m2T, srcp, dstp, exG, denG[0] + denG[1])
    xh1, xsT, s1, s2 = _tc_post_gate(accG, xh, p)
    exC, denC = _sc_alpha_scalar(s1.reshape(NPAD), s2.reshape(NPAD),
                                 srcp, dstp)
    accC = _sc_aggregate(xsT, srcp, dstp, exC, denC[0] + denC[1])
    xsn, g0 = _tc_post_conv(accC, xh1, batch_row, p)
    return _tc_readout(xsn, batch_row, batch_col, g0, p)
